# baseline (speedup 1.0000x reference)
import jax
import jax.numpy as jnp
from jax import lax
from jax.experimental import pallas as pl
from jax.experimental.pallas import tpu as pltpu
from jax.experimental.pallas import tpu_sc as plsc

N_NODES = 10000
F_IN = 128
E_HID = 16
N_REL = 11
RN = N_REL * N_NODES
NE = 650000
GR = N_NODES // 8

NUM_CORES = 2
NUM_SUBCORES = 16

CHUNK = 256
STEPS = 160
DEG_BATCH = 8
EDGES_PER_TILE = CHUNK * STEPS
NE_PAD = NUM_SUBCORES * EDGES_PER_TILE

HALF = RN // 2
ACC_H = 55040
ROWS_PER_TILE = ACC_H // NUM_SUBCORES


def _sc_agg_body(colrow_hbm, table_hbm, acc_out,
                 crbuf, rows_v, onesbuf, accv, degv,
                 acc_sh, deg_sh, sem_cr0, sem_cr1, sem_g0, sem_g1):
    sem_cr = (sem_cr0, sem_cr1)
    sem_g = (sem_g0, sem_g1)
    c = lax.axis_index("c")
    s = lax.axis_index("s")

    def zacc(i, _):
        accv[i, :] = jnp.zeros((16,), jnp.float32)
        return 0
    lax.fori_loop(0, ROWS_PER_TILE, zacc, 0)

    def zdeg(i, _):
        degv[pl.ds(i * 16, 16)] = jnp.zeros((16,), jnp.float32)
        return 0
    lax.fori_loop(0, ROWS_PER_TILE // 16, zdeg, 0)

    def orow(i, _):
        onesbuf[pl.ds(i * 16, 16)] = jnp.ones((16,), jnp.float32)
        return 0
    lax.fori_loop(0, CHUNK // 16, orow, 0)

    pltpu.sync_copy(accv, acc_sh.at[pl.ds(s * ROWS_PER_TILE, ROWS_PER_TILE), :])
    pltpu.sync_copy(degv, deg_sh.at[pl.ds(s * ROWS_PER_TILE, ROWS_PER_TILE)])
    plsc.subcore_barrier()

    rplane = 1 + c

    def start_cr(gk, slot):
        pltpu.async_copy(colrow_hbm.at[gk], crbuf.at[slot], sem_cr[slot])

    def wait_cr(gk, slot):
        pltpu.make_async_copy(colrow_hbm.at[gk], crbuf.at[slot],
                              sem_cr[slot]).wait()

    def start_g(slot):
        pltpu.async_copy(table_hbm.at[crbuf.at[slot, 0]],
                         rows_v.at[slot], sem_g[slot])

    def wait_g(slot):
        pltpu.make_async_copy(table_hbm.at[crbuf.at[slot, 0]],
                              rows_v.at[slot], sem_g[slot]).wait()

    base = s * STEPS
    start_cr(base, 0)
    start_cr(base + 1, 1)
    wait_cr(base, 0)
    start_g(0)

    def half_iter(i, slot, nslot):
        wait_g(slot)
        wait_cr(base + i + 1, nslot)
        start_g(nslot)
        pltpu.sync_copy(rows_v.at[slot],
                        acc_sh.at[crbuf.at[slot, rplane]], add=True)
        pltpu.sync_copy(onesbuf, deg_sh.at[crbuf.at[slot, rplane]], add=True)
        start_cr(base + i + 2, slot)

    def step2(i2, _):
        half_iter(2 * i2, 0, 1)
        half_iter(2 * i2 + 1, 1, 0)
        return 0
    lax.fori_loop(0, STEPS // 2, step2, 0)
    wait_g(0)
    wait_cr(base + STEPS + 1, 1)
    plsc.subcore_barrier()

    pltpu.sync_copy(acc_sh.at[pl.ds(s * ROWS_PER_TILE, ROWS_PER_TILE), :], accv)
    pltpu.sync_copy(deg_sh.at[pl.ds(s * ROWS_PER_TILE, ROWS_PER_TILE)], degv)

    def scale(g, _):
        inv = 1.0 / jnp.maximum(degv[pl.ds(g * 16, 16)], 1.0)
        for j in range(16):
            r = g * 16 + j
            accv[r, :] = accv[r, :] * inv[j]
        return 0
    lax.fori_loop(0, ROWS_PER_TILE // 16, scale, 0)

    gbase = c * HALF + s * ROWS_PER_TILE
    nreal = HALF - 15 * ROWS_PER_TILE

    @pl.when(s < NUM_SUBCORES - 1)
    def _():
        pltpu.sync_copy(accv, acc_out.at[pl.ds(gbase, ROWS_PER_TILE), :])

    @pl.when(s == NUM_SUBCORES - 1)
    def _():
        pltpu.sync_copy(accv.at[pl.ds(0, nreal), :],
                        acc_out.at[pl.ds(gbase, nreal), :])


_sc_agg = pl.kernel(
    _sc_agg_body,
    out_type=jax.ShapeDtypeStruct((RN, E_HID), jnp.float32),
    name="sc_edge_agg",
    mesh=plsc.VectorSubcoreMesh(core_axis_name="c", subcore_axis_name="s"),
    scratch_types=[
        pltpu.VMEM((2, 3, CHUNK), jnp.int32),
        pltpu.VMEM((2, CHUNK, E_HID), jnp.float32),
        pltpu.VMEM((CHUNK,), jnp.float32),
        pltpu.VMEM((ROWS_PER_TILE, E_HID), jnp.float32),
        pltpu.VMEM((ROWS_PER_TILE,), jnp.float32),
        pltpu.VMEM_SHARED((ACC_H, E_HID), jnp.float32),
        pltpu.VMEM_SHARED((ACC_H,), jnp.float32),
        pltpu.SemaphoreType.DMA,
        pltpu.SemaphoreType.DMA,
        pltpu.SemaphoreType.DMA,
        pltpu.SemaphoreType.DMA,
    ],
    compiler_params=pltpu.CompilerParams(use_tc_tiling_on_sc=False,
                                         needs_layout_passes=False),
)


def _proj_body(f_ref, w_ref, o_ref):
    o_ref[...] = jnp.dot(f_ref[...], w_ref[0],
                         preferred_element_type=jnp.float32)[None]


_proj1 = pl.pallas_call(
    _proj_body,
    grid=(N_REL,),
    in_specs=[
        pl.BlockSpec((GR, 8 * F_IN), lambda r: (0, 0)),
        pl.BlockSpec((1, 8 * F_IN, 128), lambda r: (r, 0, 0)),
    ],
    out_specs=pl.BlockSpec((1, GR, 128), lambda r: (r, 0, 0)),
    out_shape=jax.ShapeDtypeStruct((N_REL, GR, 128), jnp.float32),
)


def _mid_body(acc_ref, b1_ref, w2_ref, p2_ref):
    h = jnp.maximum(jnp.sum(acc_ref[...], axis=0) + b1_ref[...], 0.0)
    p2_ref[...] = jax.lax.dot_general(
        w2_ref[...], h, (((2,), (1,)), ((), ())),
        preferred_element_type=jnp.float32).transpose(0, 2, 1)


def _mid_body2(acc_ref, b1_ref, w2_ref, p2_ref):
    h = jnp.maximum(jnp.sum(acc_ref[...], axis=0) + b1_ref[...], 0.0)
    p2_ref[...] = jnp.einsum("nk,rkl->rnl", h, w2_ref[...],
                             preferred_element_type=jnp.float32)


_mid = pl.pallas_call(
    _mid_body2,
    grid=(1,),
    in_specs=[
        pl.BlockSpec((N_REL, GR, 128), lambda i: (0, 0, 0)),
        pl.BlockSpec((1, 128), lambda i: (0, 0)),
        pl.BlockSpec((N_REL, 128, 128), lambda i: (0, 0, 0)),
    ],
    out_specs=pl.BlockSpec((N_REL, GR, 128), lambda i: (0, 0, 0)),
    out_shape=jax.ShapeDtypeStruct((N_REL, GR, 128), jnp.float32),
)


def _final_body(acc_ref, b2_ref, o_ref):
    o_ref[...] = jnp.sum(acc_ref[...], axis=0) + b2_ref[...]


_final = pl.pallas_call(
    _final_body,
    grid=(1,),
    in_specs=[
        pl.BlockSpec((N_REL, GR, 128), lambda i: (0, 0, 0)),
        pl.BlockSpec((1, 128), lambda i: (0, 0)),
    ],
    out_specs=pl.BlockSpec((GR, 128), lambda i: (0, 0)),
    out_shape=jax.ShapeDtypeStruct((GR, 128), jnp.float32),
)


def _halves_to_dense(acc_flat):
    return acc_flat.reshape(N_REL, GR, 128)


def kernel(features, fr, to, rel, weights1, weights2, bias1, bias2):
    fr = fr.astype(jnp.int32)
    to = to.astype(jnp.int32)
    rel = rel.astype(jnp.int32)
    row = rel * N_NODES + fr
    col = rel * N_NODES + to
    npad = NE_PAD - NE
    col_pad = jnp.concatenate([col, jnp.zeros((npad,), jnp.int32)])
    row_pad = jnp.concatenate([row, jnp.full((npad,), RN, jnp.int32)])
    dummy = HALF + (jnp.arange(NE_PAD, dtype=jnp.int32) % (ACC_H - HALF))
    row0 = jnp.where(row_pad < HALF, row_pad, dummy)
    r1 = row_pad - HALF
    row1 = jnp.where((r1 >= 0) & (r1 < HALF), r1, dummy)
    nchunks = NE_PAD // CHUNK
    colrow = jnp.stack([col_pad.reshape(nchunks, CHUNK),
                        row0.reshape(nchunks, CHUNK),
                        row1.reshape(nchunks, CHUNK)], axis=1)
    colrow = jnp.concatenate(
        [colrow, jnp.zeros((2, 3, CHUNK), jnp.int32)], axis=0)

    eye8 = jnp.eye(8, dtype=jnp.float32)
    w1big = (eye8[None, :, None, :, None]
             * weights1[:, None, :, None, :]).reshape(N_REL, 8 * F_IN, 128)
    w2big = (eye8[None, :, None, :, None]
             * weights2[:, None, :, None, :]).reshape(N_REL, 128, 128)
    b1_lane = jnp.tile(bias1, 8).reshape(1, 128)
    b2_lane = jnp.tile(bias2, 8).reshape(1, 128)

    f_g = features.reshape(GR, 8 * F_IN)
    p1 = _proj1(f_g, w1big).reshape(RN, E_HID)
    acc1 = _sc_agg(colrow, p1)
    p2 = _mid(_halves_to_dense(acc1), b1_lane, w2big).reshape(RN, E_HID)
    acc2 = _sc_agg(colrow, p2)
    out = _final(_halves_to_dense(acc2), b2_lane)
    return out.reshape(N_NODES, E_HID)


# speedup vs baseline: 1.0348x; 1.0348x over previous
import jax
import jax.numpy as jnp
from jax import lax
from jax.experimental import pallas as pl
from jax.experimental.pallas import tpu as pltpu
from jax.experimental.pallas import tpu_sc as plsc

N_NODES = 10000
F_IN = 128
E_HID = 16
N_REL = 11
RN = N_REL * N_NODES
NE = 650000
GR = N_NODES // 8

NUM_CORES = 2
NUM_SUBCORES = 16

CHUNK = 256
STEPS = 160
DEG_BATCH = 8
EDGES_PER_TILE = CHUNK * STEPS
NE_PAD = NUM_SUBCORES * EDGES_PER_TILE

HALF = RN // 2
ACC_H = 55040
ROWS_PER_TILE = ACC_H // NUM_SUBCORES


def _sc_agg_body(colrow_hbm, table_hbm, acc_out,
                 crbuf, rowbuf, rows_v, onesbuf, accv, degv,
                 acc_sh, deg_sh, sem_cr0, sem_cr1, sem_g0, sem_g1):
    sem_cr = (sem_cr0, sem_cr1)
    sem_g = (sem_g0, sem_g1)
    c = lax.axis_index("c")
    s = lax.axis_index("s")

    def zacc(i, _):
        accv[i, :] = jnp.zeros((16,), jnp.float32)
        return 0
    lax.fori_loop(0, ROWS_PER_TILE, zacc, 0)

    def zdeg(i, _):
        degv[pl.ds(i * 16, 16)] = jnp.zeros((16,), jnp.float32)
        return 0
    lax.fori_loop(0, ROWS_PER_TILE // 16, zdeg, 0)

    def orow(i, _):
        onesbuf[pl.ds(i * 16, 16)] = jnp.ones((16,), jnp.float32)
        return 0
    lax.fori_loop(0, CHUNK // 16, orow, 0)

    pltpu.sync_copy(accv, acc_sh.at[pl.ds(s * ROWS_PER_TILE, ROWS_PER_TILE), :])
    pltpu.sync_copy(degv, deg_sh.at[pl.ds(s * ROWS_PER_TILE, ROWS_PER_TILE)])
    plsc.subcore_barrier()

    lo = c * HALF
    lanes = lax.iota(jnp.int32, 16)

    def start_cr(gk, slot):
        pltpu.async_copy(colrow_hbm.at[gk], crbuf.at[slot], sem_cr[slot])

    def wait_cr(gk, slot):
        pltpu.make_async_copy(colrow_hbm.at[gk], crbuf.at[slot],
                              sem_cr[slot]).wait()

    def start_g(slot):
        pltpu.async_copy(table_hbm.at[crbuf.at[slot, 0]],
                         rows_v.at[slot], sem_g[slot])

    def wait_g(slot):
        pltpu.make_async_copy(table_hbm.at[crbuf.at[slot, 0]],
                              rows_v.at[slot], sem_g[slot]).wait()

    base = s * STEPS
    start_cr(base, 0)
    start_cr(base + 1, 1)
    wait_cr(base, 0)
    start_g(0)

    def half_iter(i, slot, nslot):
        wait_g(slot)

        def remap(g, _):
            v = crbuf[slot, 1, pl.ds(g * 16, 16)] - lo
            ok = (v >= 0) & (v < HALF)
            rowbuf[slot, pl.ds(g * 16, 16)] = jnp.where(ok, v, HALF + lanes)
            return 0
        lax.fori_loop(0, CHUNK // 16, remap, 0)

        start_cr(base + i + 2, slot)
        wait_cr(base + i + 1, nslot)
        start_g(nslot)
        pltpu.sync_copy(rows_v.at[slot], acc_sh.at[rowbuf.at[slot]], add=True)
        pltpu.sync_copy(onesbuf, deg_sh.at[rowbuf.at[slot]], add=True)

    def step2(i2, _):
        half_iter(2 * i2, 0, 1)
        half_iter(2 * i2 + 1, 1, 0)
        return 0
    lax.fori_loop(0, STEPS // 2, step2, 0)
    wait_g(0)
    wait_cr(base + STEPS + 1, 1)
    plsc.subcore_barrier()

    pltpu.sync_copy(acc_sh.at[pl.ds(s * ROWS_PER_TILE, ROWS_PER_TILE), :], accv)
    pltpu.sync_copy(deg_sh.at[pl.ds(s * ROWS_PER_TILE, ROWS_PER_TILE)], degv)

    def scale(g, _):
        inv = 1.0 / jnp.maximum(degv[pl.ds(g * 16, 16)], 1.0)
        for j in range(16):
            r = g * 16 + j
            accv[r, :] = accv[r, :] * inv[j]
        return 0
    lax.fori_loop(0, ROWS_PER_TILE // 16, scale, 0)

    gbase = c * HALF + s * ROWS_PER_TILE
    nreal = HALF - 15 * ROWS_PER_TILE

    @pl.when(s < NUM_SUBCORES - 1)
    def _():
        pltpu.sync_copy(accv, acc_out.at[pl.ds(gbase, ROWS_PER_TILE), :])

    @pl.when(s == NUM_SUBCORES - 1)
    def _():
        pltpu.sync_copy(accv.at[pl.ds(0, nreal), :],
                        acc_out.at[pl.ds(gbase, nreal), :])


_sc_agg = pl.kernel(
    _sc_agg_body,
    out_type=jax.ShapeDtypeStruct((RN, E_HID), jnp.float32),
    name="sc_edge_agg",
    mesh=plsc.VectorSubcoreMesh(core_axis_name="c", subcore_axis_name="s"),
    scratch_types=[
        pltpu.VMEM((2, 2, CHUNK), jnp.int32),
        pltpu.VMEM((2, CHUNK), jnp.int32),
        pltpu.VMEM((2, CHUNK, E_HID), jnp.float32),
        pltpu.VMEM((CHUNK,), jnp.float32),
        pltpu.VMEM((ROWS_PER_TILE, E_HID), jnp.float32),
        pltpu.VMEM((ROWS_PER_TILE,), jnp.float32),
        pltpu.VMEM_SHARED((ACC_H, E_HID), jnp.float32),
        pltpu.VMEM_SHARED((ACC_H,), jnp.float32),
        pltpu.SemaphoreType.DMA,
        pltpu.SemaphoreType.DMA,
        pltpu.SemaphoreType.DMA,
        pltpu.SemaphoreType.DMA,
    ],
    compiler_params=pltpu.CompilerParams(use_tc_tiling_on_sc=False,
                                         needs_layout_passes=False),
)


def _proj_body(f_ref, w_ref, o_ref):
    o_ref[...] = jnp.dot(f_ref[...], w_ref[0],
                         preferred_element_type=jnp.float32)[None]


_proj1 = pl.pallas_call(
    _proj_body,
    grid=(N_REL,),
    in_specs=[
        pl.BlockSpec((GR, 8 * F_IN), lambda r: (0, 0)),
        pl.BlockSpec((1, 8 * F_IN, 128), lambda r: (r, 0, 0)),
    ],
    out_specs=pl.BlockSpec((1, GR, 128), lambda r: (r, 0, 0)),
    out_shape=jax.ShapeDtypeStruct((N_REL, GR, 128), jnp.float32),
)


def _mid_body(acc_ref, b1_ref, w2_ref, p2_ref):
    h = jnp.maximum(jnp.sum(acc_ref[...], axis=0) + b1_ref[...], 0.0)
    p2_ref[...] = jax.lax.dot_general(
        w2_ref[...], h, (((2,), (1,)), ((), ())),
        preferred_element_type=jnp.float32).transpose(0, 2, 1)


def _mid_body2(acc_ref, b1_ref, w2_ref, p2_ref):
    h = jnp.maximum(jnp.sum(acc_ref[...], axis=0) + b1_ref[...], 0.0)
    p2_ref[...] = jnp.einsum("nk,rkl->rnl", h, w2_ref[...],
                             preferred_element_type=jnp.float32)


_mid = pl.pallas_call(
    _mid_body2,
    grid=(1,),
    in_specs=[
        pl.BlockSpec((N_REL, GR, 128), lambda i: (0, 0, 0)),
        pl.BlockSpec((1, 128), lambda i: (0, 0)),
        pl.BlockSpec((N_REL, 128, 128), lambda i: (0, 0, 0)),
    ],
    out_specs=pl.BlockSpec((N_REL, GR, 128), lambda i: (0, 0, 0)),
    out_shape=jax.ShapeDtypeStruct((N_REL, GR, 128), jnp.float32),
)


def _final_body(acc_ref, b2_ref, o_ref):
    o_ref[...] = jnp.sum(acc_ref[...], axis=0) + b2_ref[...]


_final = pl.pallas_call(
    _final_body,
    grid=(1,),
    in_specs=[
        pl.BlockSpec((N_REL, GR, 128), lambda i: (0, 0, 0)),
        pl.BlockSpec((1, 128), lambda i: (0, 0)),
    ],
    out_specs=pl.BlockSpec((GR, 128), lambda i: (0, 0)),
    out_shape=jax.ShapeDtypeStruct((GR, 128), jnp.float32),
)


def _halves_to_dense(acc_flat):
    return acc_flat.reshape(N_REL, GR, 128)


def kernel(features, fr, to, rel, weights1, weights2, bias1, bias2):
    fr = fr.astype(jnp.int32)
    to = to.astype(jnp.int32)
    rel = rel.astype(jnp.int32)
    row = rel * N_NODES + fr
    col = rel * N_NODES + to
    npad = NE_PAD - NE
    col_pad = jnp.concatenate([col, jnp.zeros((npad,), jnp.int32)])
    row_pad = jnp.concatenate([row, jnp.full((npad,), RN, jnp.int32)])
    nchunks = NE_PAD // CHUNK
    colrow = jnp.stack([col_pad.reshape(nchunks, CHUNK),
                        row_pad.reshape(nchunks, CHUNK)], axis=1)
    colrow = jnp.concatenate(
        [colrow, jnp.zeros((2, 2, CHUNK), jnp.int32)], axis=0)

    eye8 = jnp.eye(8, dtype=jnp.float32)
    w1big = (eye8[None, :, None, :, None]
             * weights1[:, None, :, None, :]).reshape(N_REL, 8 * F_IN, 128)
    w2big = (eye8[None, :, None, :, None]
             * weights2[:, None, :, None, :]).reshape(N_REL, 128, 128)
    b1_lane = jnp.tile(bias1, 8).reshape(1, 128)
    b2_lane = jnp.tile(bias2, 8).reshape(1, 128)

    f_g = features.reshape(GR, 8 * F_IN)
    p1 = _proj1(f_g, w1big).reshape(RN, E_HID)
    acc1 = _sc_agg(colrow, p1)
    p2 = _mid(_halves_to_dense(acc1), b1_lane, w2big).reshape(RN, E_HID)
    acc2 = _sc_agg(colrow, p2)
    out = _final(_halves_to_dense(acc2), b2_lane)
    return out.reshape(N_NODES, E_HID)
